# baseline (speedup 1.0000x reference)
import functools

import jax
import jax.numpy as jnp
from jax import lax
from jax.experimental import pallas as pl
from jax.experimental.pallas import tpu as pltpu
from jax.experimental.pallas import tpu_sc as plsc

_N = 10000
_E = 160000
_D = 128
_DE = 16
_H = 16
_G = 64

_NC = 2
_NS = 16
_NW = _NC * _NS
_CHUNK = 128
_CPT = 40
_EPAD = _NW * _CPT * _CHUNK
_NROWS = 10112
_RPT = _NROWS // _NS
_EB = 512


def _bf16_dot(a, b):
    return jnp.dot(a.astype(jnp.bfloat16), b.astype(jnp.bfloat16),
                   preferred_element_type=jnp.float32)


def _bround(a):
    return a.astype(jnp.bfloat16).astype(jnp.float32)


def _einsum_body(din, xg_ref, eat_ref, w_ref, b_ref, msg_ref):
    wt = _bround(_bf16_dot(w_ref[...], eat_ref[...]) + b_ref[:, 0:1])
    xt = _bround(jnp.transpose(xg_ref[...]))
    acc = xt[0:1, :] * wt[0:_H, :]
    for i in range(1, din):
        acc = acc + xt[i:i + 1, :] * wt[i * _H:(i + 1) * _H, :]
    msg_ref[...] = jnp.transpose(acc)


def _tc_einsum(din, xg, eat, lin_w, lin_b8):
    return pl.pallas_call(
        functools.partial(_einsum_body, din),
        grid=(_EPAD // _EB,),
        in_specs=[
            pl.BlockSpec((_EB, din), lambda i: (i, 0)),
            pl.BlockSpec((_DE, _EB), lambda i: (0, i)),
            pl.BlockSpec((din * _H, _DE), lambda i: (0, 0)),
            pl.BlockSpec((din * _H, 8), lambda i: (0, 0)),
        ],
        out_specs=pl.BlockSpec((_EB, _H), lambda i: (i, 0)),
        out_shape=jax.ShapeDtypeStruct((_EPAD, _H), jnp.float32),
    )(xg, eat, lin_w, lin_b8)


def _node_body(rnd, p_ref, h_ref, root_ref, b_ref, out_ref):
    h = jax.nn.relu(p_ref[0, :_N] + p_ref[1, :_N]
                    + _bf16_dot(h_ref[...], root_ref[...]) + b_ref[...])
    out_ref[...] = _bround(h) if rnd else h


def _tc_node(parts, h, root, bias, rnd):
    return pl.pallas_call(
        functools.partial(_node_body, rnd),
        out_shape=jax.ShapeDtypeStruct((_N, _H), jnp.float32),
    )(parts, h, root, bias)


def _final_body(h_ref, batch_ref,
                w1_ref, b1_ref, w2_ref, b2_ref, w3_ref, b3_ref, w4_ref, b4_ref,
                out_ref):
    seg = lax.broadcasted_iota(jnp.int32, (_G, _N), 0)
    onehot = (seg == batch_ref[0][None, :]).astype(jnp.float32)
    g = jnp.dot(onehot, h_ref[...], preferred_element_type=jnp.float32,
                precision=lax.Precision.HIGHEST)
    g = jax.nn.relu(_bf16_dot(g, w1_ref[...]) + b1_ref[...])
    g = jax.nn.relu(_bf16_dot(g, w2_ref[...]) + b2_ref[...])
    g = jax.nn.relu(_bf16_dot(g, w3_ref[...]) + b3_ref[...])
    out_ref[...] = _bf16_dot(g, w4_ref[...]) + b4_ref[...]


def _tc_final(h3, batch2d, w1, b1, w2, b2, w3, b3, w4, b4):
    return pl.pallas_call(
        _final_body,
        out_shape=jax.ShapeDtypeStruct((_G, 8), jnp.float32),
    )(h3, batch2d, w1, b1, w2, b2, w3, b3, w4, b4)


def _make_gather(dw):
    mesh = plsc.VectorSubcoreMesh(core_axis_name="c", subcore_axis_name="s")
    return functools.partial(
        pl.kernel,
        out_type=jax.ShapeDtypeStruct((_EPAD, dw), jnp.float32),
        mesh=mesh,
        scratch_types=[
            pltpu.VMEM((_CPT, _CHUNK), jnp.int32),
            pltpu.VMEM((_CHUNK, dw), jnp.float32),
            pltpu.VMEM((_CHUNK, dw), jnp.float32),
            pltpu.SemaphoreType.DMA,
            pltpu.SemaphoreType.DMA,
            pltpu.SemaphoreType.DMA,
            pltpu.SemaphoreType.DMA,
        ],
        compiler_params=pltpu.CompilerParams(use_tc_tiling_on_sc=False),
    )(_gather_body)


def _gather_body(tab_hbm, src_hbm, out_hbm,
                 src_v, rows_a, rows_b, gs_a, gs_b, ws_a, ws_b):
    cid = lax.axis_index("c")
    sid = lax.axis_index("s")
    wid = cid * _NS + sid
    pltpu.sync_copy(src_hbm.at[pl.ds(wid * _CPT, _CPT)], src_v)

    def start(j, rows, gsem):
        pltpu.async_copy(tab_hbm.at[src_v.at[j]], rows, gsem)

    def wait_g(rows, gsem):
        pltpu.make_async_copy(tab_hbm.at[src_v.at[0]], rows, gsem).wait()

    def start_w(j, rows, wsem):
        base_e = (wid * _CPT + j) * _CHUNK
        pltpu.async_copy(rows, out_hbm.at[pl.ds(base_e, _CHUNK)], wsem)

    def wait_w(rows, wsem):
        pltpu.make_async_copy(rows, out_hbm.at[pl.ds(0, _CHUNK)], wsem).wait()

    start(0, rows_a, gs_a)
    start(1, rows_b, gs_b)

    def body(jj, carry):
        j0 = 2 * jj
        wait_g(rows_a, gs_a)
        start_w(j0, rows_a, ws_a)
        wait_g(rows_b, gs_b)
        start_w(j0 + 1, rows_b, ws_b)

        @pl.when(jj < _CPT // 2 - 1)
        def _():
            wait_w(rows_a, ws_a)
            start(j0 + 2, rows_a, gs_a)
            wait_w(rows_b, ws_b)
            start(j0 + 3, rows_b, gs_b)

        return carry

    lax.fori_loop(0, _CPT // 2, body, 0)
    wait_w(rows_a, ws_a)
    wait_w(rows_b, ws_b)


def _make_scatter():
    mesh = plsc.VectorSubcoreMesh(core_axis_name="c", subcore_axis_name="s")
    return functools.partial(
        pl.kernel,
        out_type=jax.ShapeDtypeStruct((_NC, _NROWS, _H), jnp.float32),
        mesh=mesh,
        scratch_types=[
            pltpu.VMEM((_CPT, _CHUNK), jnp.int32),
            pltpu.VMEM((_CHUNK, _H), jnp.float32),
            pltpu.VMEM((_CHUNK, _H), jnp.float32),
            pltpu.VMEM((_RPT, _H), jnp.float32),
            pltpu.VMEM_SHARED((_NROWS, _H), jnp.float32),
            pltpu.SemaphoreType.DMA,
            pltpu.SemaphoreType.DMA,
            pltpu.SemaphoreType.DMA,
            pltpu.SemaphoreType.DMA,
        ],
        compiler_params=pltpu.CompilerParams(use_tc_tiling_on_sc=False),
    )(_scatter_body)


def _scatter_body(msg_hbm, dst_hbm, out_hbm,
                  dst_v, msg_a, msg_b, zb_v, acc_sh, ls_a, ls_b, ss_a, ss_b):
    cid = lax.axis_index("c")
    sid = lax.axis_index("s")
    wid = cid * _NS + sid
    pltpu.sync_copy(dst_hbm.at[pl.ds(wid * _CPT, _CPT)], dst_v)

    def start_l(j, msg, lsem):
        base_e = (wid * _CPT + j) * _CHUNK
        pltpu.async_copy(msg_hbm.at[pl.ds(base_e, _CHUNK)], msg, lsem)

    def wait_l(msg, lsem):
        pltpu.make_async_copy(msg_hbm.at[pl.ds(0, _CHUNK)], msg, lsem).wait()

    def start_s(j, msg, ssem):
        pltpu.async_copy(msg, acc_sh.at[dst_v.at[j]], ssem, add=True)

    def wait_s(msg, ssem):
        pltpu.make_async_copy(msg, acc_sh.at[dst_v.at[0]], ssem).wait()

    start_l(0, msg_a, ls_a)
    start_l(1, msg_b, ls_b)

    zeros16 = jnp.zeros((_H,), jnp.float32)

    def zinit(i, carry):
        zb_v[i, :] = zeros16
        return carry

    lax.fori_loop(0, _RPT, zinit, 0)
    pltpu.sync_copy(zb_v, acc_sh.at[pl.ds(sid * _RPT, _RPT)])
    plsc.subcore_barrier()

    def body(jj, carry):
        j0 = 2 * jj
        wait_l(msg_a, ls_a)
        start_s(j0, msg_a, ss_a)
        wait_l(msg_b, ls_b)
        start_s(j0 + 1, msg_b, ss_b)

        @pl.when(jj < _CPT // 2 - 1)
        def _():
            wait_s(msg_a, ss_a)
            start_l(j0 + 2, msg_a, ls_a)
            wait_s(msg_b, ss_b)
            start_l(j0 + 3, msg_b, ls_b)

        return carry

    lax.fori_loop(0, _CPT // 2, body, 0)
    wait_s(msg_a, ss_a)
    wait_s(msg_b, ss_b)
    plsc.subcore_barrier()
    pltpu.sync_copy(acc_sh.at[pl.ds(sid * _RPT, _RPT)], zb_v)
    pltpu.sync_copy(zb_v, out_hbm.at[cid, pl.ds(sid * _RPT, _RPT)])


@functools.cache
def _sc_kernels():
    return _make_gather(_D), _make_gather(_H), _make_scatter()


def kernel(x, edge_index, edge_attr, batch,
           lin1_w, lin1_b, root1, bias1,
           lin2_w, lin2_b, root2, bias2,
           lin3_w, lin3_b, root3, bias3,
           fc1_w, fc1_b, fc2_w, fc2_b, fc3_w, fc3_b, fc4_w, fc4_b):
    gather_x, gather_h, scatter = _sc_kernels()
    pad = _EPAD - _E
    src = jnp.concatenate([edge_index[0], jnp.zeros((pad,), jnp.int32)])
    src = src.reshape(_EPAD // _CHUNK, _CHUNK)
    dst = jnp.concatenate([edge_index[1], jnp.full((pad,), _N, jnp.int32)])
    dst = dst.reshape(_EPAD // _CHUNK, _CHUNK)
    eat = jnp.concatenate(
        [edge_attr, jnp.zeros((pad, _DE), jnp.float32)], axis=0).T

    def b8(b):
        return jnp.pad(b.reshape(-1, 1), ((0, 0), (0, 7)))

    hr = x
    h = x
    for lw, lb, rt, bs, din, last in (
            (lin1_w, lin1_b, root1, bias1, _D, False),
            (lin2_w, lin2_b, root2, bias2, _H, False),
            (lin3_w, lin3_b, root3, bias3, _H, True)):
        xg = (gather_x if din == _D else gather_h)(hr, src)
        msg = _tc_einsum(din, xg, eat, lw, b8(lb))
        parts = scatter(msg, dst)
        nxt = _tc_node(parts, h, rt, bs.reshape(1, _H), rnd=not last)
        hr, h = nxt, nxt
    out8 = _tc_final(h, batch.reshape(1, _N),
                     fc1_w.T, fc1_b.reshape(1, -1),
                     fc2_w.T, fc2_b.reshape(1, -1),
                     fc3_w.T, fc3_b.reshape(1, -1),
                     jnp.pad(fc4_w.T, ((0, 0), (0, 7))),
                     jnp.pad(fc4_b, (0, 7)).reshape(1, 8))
    return out8[:, 0]


# speedup vs baseline: 1.1611x; 1.1611x over previous
import functools

import jax
import jax.numpy as jnp
from jax import lax
from jax.experimental import pallas as pl
from jax.experimental.pallas import tpu as pltpu
from jax.experimental.pallas import tpu_sc as plsc

_N = 10000
_E = 160000
_D = 128
_DE = 16
_H = 16
_G = 64

_NC = 2
_NS = 16
_NW = _NC * _NS
_CHUNK = 128
_CPT = 40
_EPAD = _NW * _CPT * _CHUNK
_NROWS = 10112
_RPT = _NROWS // _NS
_EB = 1024


def _bf16_dot(a, b):
    return jnp.dot(a.astype(jnp.bfloat16), b.astype(jnp.bfloat16),
                   preferred_element_type=jnp.float32)


def _bround(a):
    return a.astype(jnp.bfloat16).astype(jnp.float32)


def _einsum_body(din, xg_ref, eat_ref, w_ref, b_ref, msg_ref):
    wt = _bround(_bf16_dot(w_ref[...], eat_ref[...]) + b_ref[:, 0:1])
    xt = _bround(jnp.transpose(xg_ref[...]))
    acc = xt[0:1, :] * wt[0:_H, :]
    for i in range(1, din):
        acc = acc + xt[i:i + 1, :] * wt[i * _H:(i + 1) * _H, :]
    msg_ref[...] = jnp.transpose(acc)


def _tc_einsum(din, xg, eat, lin_w, lin_b8):
    return pl.pallas_call(
        functools.partial(_einsum_body, din),
        grid=(_EPAD // _EB,),
        in_specs=[
            pl.BlockSpec((_EB, din), lambda i: (i, 0)),
            pl.BlockSpec((_DE, _EB), lambda i: (0, i)),
            pl.BlockSpec((din * _H, _DE), lambda i: (0, 0)),
            pl.BlockSpec((din * _H, 8), lambda i: (0, 0)),
        ],
        out_specs=pl.BlockSpec((_EB, _H), lambda i: (i, 0)),
        out_shape=jax.ShapeDtypeStruct((_EPAD, _H), jnp.float32),
    )(xg, eat, lin_w, lin_b8)


def _node_body(rnd, p_ref, h_ref, root_ref, b_ref, out_ref):
    h = jax.nn.relu(p_ref[0, :_N] + p_ref[1, :_N]
                    + _bf16_dot(h_ref[...], root_ref[...]) + b_ref[...])
    out_ref[...] = _bround(h) if rnd else h


def _tc_node(parts, h, root, bias, rnd):
    return pl.pallas_call(
        functools.partial(_node_body, rnd),
        out_shape=jax.ShapeDtypeStruct((_N, _H), jnp.float32),
    )(parts, h, root, bias)


def _final_body(h_ref, batch_ref,
                w1_ref, b1_ref, w2_ref, b2_ref, w3_ref, b3_ref, w4_ref, b4_ref,
                out_ref):
    seg = lax.broadcasted_iota(jnp.int32, (_G, _N), 0)
    onehot = (seg == batch_ref[0][None, :]).astype(jnp.float32)
    g = jnp.dot(onehot, h_ref[...], preferred_element_type=jnp.float32,
                precision=lax.Precision.HIGHEST)
    g = jax.nn.relu(_bf16_dot(g, w1_ref[...]) + b1_ref[...])
    g = jax.nn.relu(_bf16_dot(g, w2_ref[...]) + b2_ref[...])
    g = jax.nn.relu(_bf16_dot(g, w3_ref[...]) + b3_ref[...])
    out_ref[...] = _bf16_dot(g, w4_ref[...]) + b4_ref[...]


def _tc_final(h3, batch2d, w1, b1, w2, b2, w3, b3, w4, b4):
    return pl.pallas_call(
        _final_body,
        out_shape=jax.ShapeDtypeStruct((_G, 8), jnp.float32),
    )(h3, batch2d, w1, b1, w2, b2, w3, b3, w4, b4)


def _make_gather(dw):
    mesh = plsc.VectorSubcoreMesh(core_axis_name="c", subcore_axis_name="s")
    return functools.partial(
        pl.kernel,
        out_type=jax.ShapeDtypeStruct((_EPAD, dw), jnp.float32),
        mesh=mesh,
        scratch_types=[
            pltpu.VMEM((_CPT, _CHUNK), jnp.int32),
            pltpu.VMEM((_CHUNK, dw), jnp.float32),
            pltpu.VMEM((_CHUNK, dw), jnp.float32),
            pltpu.SemaphoreType.DMA,
            pltpu.SemaphoreType.DMA,
            pltpu.SemaphoreType.DMA,
            pltpu.SemaphoreType.DMA,
        ],
        compiler_params=pltpu.CompilerParams(use_tc_tiling_on_sc=False),
    )(_gather_body)


def _gather_body(tab_hbm, src_hbm, out_hbm,
                 src_v, rows_a, rows_b, gs_a, gs_b, ws_a, ws_b):
    cid = lax.axis_index("c")
    sid = lax.axis_index("s")
    wid = cid * _NS + sid
    pltpu.sync_copy(src_hbm.at[pl.ds(wid * _CPT, _CPT)], src_v)

    def start(j, rows, gsem):
        pltpu.async_copy(tab_hbm.at[src_v.at[j]], rows, gsem)

    def wait_g(rows, gsem):
        pltpu.make_async_copy(tab_hbm.at[src_v.at[0]], rows, gsem).wait()

    def start_w(j, rows, wsem):
        base_e = (wid * _CPT + j) * _CHUNK
        pltpu.async_copy(rows, out_hbm.at[pl.ds(base_e, _CHUNK)], wsem)

    def wait_w(rows, wsem):
        pltpu.make_async_copy(rows, out_hbm.at[pl.ds(0, _CHUNK)], wsem).wait()

    start(0, rows_a, gs_a)
    start(1, rows_b, gs_b)

    def body(jj, carry):
        j0 = 2 * jj
        wait_g(rows_a, gs_a)
        start_w(j0, rows_a, ws_a)
        wait_g(rows_b, gs_b)
        start_w(j0 + 1, rows_b, ws_b)

        @pl.when(jj < _CPT // 2 - 1)
        def _():
            wait_w(rows_a, ws_a)
            start(j0 + 2, rows_a, gs_a)
            wait_w(rows_b, ws_b)
            start(j0 + 3, rows_b, gs_b)

        return carry

    lax.fori_loop(0, _CPT // 2, body, 0)
    wait_w(rows_a, ws_a)
    wait_w(rows_b, ws_b)


def _make_scatter():
    mesh = plsc.VectorSubcoreMesh(core_axis_name="c", subcore_axis_name="s")
    return functools.partial(
        pl.kernel,
        out_type=jax.ShapeDtypeStruct((_NC, _NROWS, _H), jnp.float32),
        mesh=mesh,
        scratch_types=[
            pltpu.VMEM((_CPT, _CHUNK), jnp.int32),
            pltpu.VMEM((_CHUNK, _H), jnp.float32),
            pltpu.VMEM((_CHUNK, _H), jnp.float32),
            pltpu.VMEM((_RPT, _H), jnp.float32),
            pltpu.VMEM_SHARED((_NROWS, _H), jnp.float32),
            pltpu.SemaphoreType.DMA,
            pltpu.SemaphoreType.DMA,
            pltpu.SemaphoreType.DMA,
            pltpu.SemaphoreType.DMA,
        ],
        compiler_params=pltpu.CompilerParams(use_tc_tiling_on_sc=False),
    )(_scatter_body)


def _scatter_body(msg_hbm, dst_hbm, out_hbm,
                  dst_v, msg_a, msg_b, zb_v, acc_sh, ls_a, ls_b, ss_a, ss_b):
    cid = lax.axis_index("c")
    sid = lax.axis_index("s")
    wid = cid * _NS + sid
    pltpu.sync_copy(dst_hbm.at[pl.ds(wid * _CPT, _CPT)], dst_v)

    def start_l(j, msg, lsem):
        base_e = (wid * _CPT + j) * _CHUNK
        pltpu.async_copy(msg_hbm.at[pl.ds(base_e, _CHUNK)], msg, lsem)

    def wait_l(msg, lsem):
        pltpu.make_async_copy(msg_hbm.at[pl.ds(0, _CHUNK)], msg, lsem).wait()

    def start_s(j, msg, ssem):
        pltpu.async_copy(msg, acc_sh.at[dst_v.at[j]], ssem, add=True)

    def wait_s(msg, ssem):
        pltpu.make_async_copy(msg, acc_sh.at[dst_v.at[0]], ssem).wait()

    start_l(0, msg_a, ls_a)
    start_l(1, msg_b, ls_b)

    zeros16 = jnp.zeros((_H,), jnp.float32)

    def zinit(i, carry):
        zb_v[i, :] = zeros16
        return carry

    lax.fori_loop(0, _RPT, zinit, 0)
    pltpu.sync_copy(zb_v, acc_sh.at[pl.ds(sid * _RPT, _RPT)])
    plsc.subcore_barrier()

    def body(jj, carry):
        j0 = 2 * jj
        wait_l(msg_a, ls_a)
        start_s(j0, msg_a, ss_a)
        wait_l(msg_b, ls_b)
        start_s(j0 + 1, msg_b, ss_b)

        @pl.when(jj < _CPT // 2 - 1)
        def _():
            wait_s(msg_a, ss_a)
            start_l(j0 + 2, msg_a, ls_a)
            wait_s(msg_b, ss_b)
            start_l(j0 + 3, msg_b, ls_b)

        return carry

    lax.fori_loop(0, _CPT // 2, body, 0)
    wait_s(msg_a, ss_a)
    wait_s(msg_b, ss_b)
    plsc.subcore_barrier()
    pltpu.sync_copy(acc_sh.at[pl.ds(sid * _RPT, _RPT)], zb_v)
    pltpu.sync_copy(zb_v, out_hbm.at[cid, pl.ds(sid * _RPT, _RPT)])


@functools.cache
def _sc_kernels():
    return _make_gather(_D), _make_gather(_H), _make_scatter()


def kernel(x, edge_index, edge_attr, batch,
           lin1_w, lin1_b, root1, bias1,
           lin2_w, lin2_b, root2, bias2,
           lin3_w, lin3_b, root3, bias3,
           fc1_w, fc1_b, fc2_w, fc2_b, fc3_w, fc3_b, fc4_w, fc4_b):
    gather_x, gather_h, scatter = _sc_kernels()
    pad = _EPAD - _E
    src = jnp.concatenate([edge_index[0], jnp.zeros((pad,), jnp.int32)])
    src = src.reshape(_EPAD // _CHUNK, _CHUNK)
    dst = jnp.concatenate([edge_index[1], jnp.full((pad,), _N, jnp.int32)])
    dst = dst.reshape(_EPAD // _CHUNK, _CHUNK)
    eat = jnp.concatenate(
        [edge_attr, jnp.zeros((pad, _DE), jnp.float32)], axis=0).T

    def b8(b):
        return jnp.pad(b.reshape(-1, 1), ((0, 0), (0, 7)))

    hr = x
    h = x
    for lw, lb, rt, bs, din, last in (
            (lin1_w, lin1_b, root1, bias1, _D, False),
            (lin2_w, lin2_b, root2, bias2, _H, False),
            (lin3_w, lin3_b, root3, bias3, _H, True)):
        xg = (gather_x if din == _D else gather_h)(hr, src)
        msg = _tc_einsum(din, xg, eat, lw, b8(lb))
        parts = scatter(msg, dst)
        nxt = _tc_node(parts, h, rt, bs.reshape(1, _H), rnd=not last)
        hr, h = nxt, nxt
    out8 = _tc_final(h, batch.reshape(1, _N),
                     fc1_w.T, fc1_b.reshape(1, -1),
                     fc2_w.T, fc2_b.reshape(1, -1),
                     fc3_w.T, fc3_b.reshape(1, -1),
                     jnp.pad(fc4_w.T, ((0, 0), (0, 7))),
                     jnp.pad(fc4_b, (0, 7)).reshape(1, 8))
    return out8[:, 0]


# speedup vs baseline: 1.2857x; 1.1073x over previous
import functools

import jax
import jax.numpy as jnp
from jax import lax
from jax.experimental import pallas as pl
from jax.experimental.pallas import tpu as pltpu
from jax.experimental.pallas import tpu_sc as plsc

_N = 10000
_E = 160000
_D = 128
_DE = 16
_H = 16
_G = 64

_NC = 2
_NS = 16
_NW = _NC * _NS
_CHUNK = 128
_CPT = 40
_EPAD = _NW * _CPT * _CHUNK
_NROWS = 10112
_RPT = _NROWS // _NS
_EB = 2048


def _bf16_dot(a, b):
    return jnp.dot(a.astype(jnp.bfloat16), b.astype(jnp.bfloat16),
                   preferred_element_type=jnp.float32)


def _bround(a):
    return a.astype(jnp.bfloat16).astype(jnp.float32)


def _einsum_body(din, xg_ref, eat_ref, w_ref, b_ref, msg_ref):
    wt = _bround(_bf16_dot(w_ref[...], eat_ref[...]) + b_ref[:, 0:1])
    xt = _bround(jnp.transpose(xg_ref[...]))
    acc = xt[0:1, :] * wt[0:_H, :]
    for i in range(1, din):
        acc = acc + xt[i:i + 1, :] * wt[i * _H:(i + 1) * _H, :]
    msg_ref[...] = jnp.transpose(acc)


def _tc_einsum(din, xg, eat, lin_w, lin_b8):
    return pl.pallas_call(
        functools.partial(_einsum_body, din),
        grid=(_EPAD // _EB,),
        in_specs=[
            pl.BlockSpec((_EB, din), lambda i: (i, 0)),
            pl.BlockSpec((_DE, _EB), lambda i: (0, i)),
            pl.BlockSpec((din * _H, _DE), lambda i: (0, 0)),
            pl.BlockSpec((din * _H, 8), lambda i: (0, 0)),
        ],
        out_specs=pl.BlockSpec((_EB, _H), lambda i: (i, 0)),
        out_shape=jax.ShapeDtypeStruct((_EPAD, _H), jnp.float32),
    )(xg, eat, lin_w, lin_b8)


def _node_body(rnd, p_ref, h_ref, root_ref, b_ref, out_ref):
    h = jax.nn.relu(p_ref[0, :_N] + p_ref[1, :_N]
                    + _bf16_dot(h_ref[...], root_ref[...]) + b_ref[...])
    out_ref[...] = _bround(h) if rnd else h


def _tc_node(parts, h, root, bias, rnd):
    return pl.pallas_call(
        functools.partial(_node_body, rnd),
        out_shape=jax.ShapeDtypeStruct((_N, _H), jnp.float32),
    )(parts, h, root, bias)


def _final_body(h_ref, batch_ref,
                w1_ref, b1_ref, w2_ref, b2_ref, w3_ref, b3_ref, w4_ref, b4_ref,
                out_ref):
    seg = lax.broadcasted_iota(jnp.int32, (_G, _N), 0)
    onehot = (seg == batch_ref[0][None, :]).astype(jnp.float32)
    g = jnp.dot(onehot, h_ref[...], preferred_element_type=jnp.float32,
                precision=lax.Precision.HIGHEST)
    g = jax.nn.relu(_bf16_dot(g, w1_ref[...]) + b1_ref[...])
    g = jax.nn.relu(_bf16_dot(g, w2_ref[...]) + b2_ref[...])
    g = jax.nn.relu(_bf16_dot(g, w3_ref[...]) + b3_ref[...])
    out_ref[...] = _bf16_dot(g, w4_ref[...]) + b4_ref[...]


def _tc_final(h3, batch2d, w1, b1, w2, b2, w3, b3, w4, b4):
    return pl.pallas_call(
        _final_body,
        out_shape=jax.ShapeDtypeStruct((_G, 8), jnp.float32),
    )(h3, batch2d, w1, b1, w2, b2, w3, b3, w4, b4)


def _make_gather(dw):
    mesh = plsc.VectorSubcoreMesh(core_axis_name="c", subcore_axis_name="s")
    return functools.partial(
        pl.kernel,
        out_type=jax.ShapeDtypeStruct((_EPAD, dw), jnp.float32),
        mesh=mesh,
        scratch_types=[
            pltpu.VMEM((_CPT, _CHUNK), jnp.int32),
            pltpu.VMEM((_CHUNK, dw), jnp.float32),
            pltpu.VMEM((_CHUNK, dw), jnp.float32),
            pltpu.SemaphoreType.DMA,
            pltpu.SemaphoreType.DMA,
            pltpu.SemaphoreType.DMA,
            pltpu.SemaphoreType.DMA,
        ],
        compiler_params=pltpu.CompilerParams(use_tc_tiling_on_sc=False),
    )(_gather_body)


def _gather_body(tab_hbm, src_hbm, out_hbm,
                 src_v, rows_a, rows_b, gs_a, gs_b, ws_a, ws_b):
    cid = lax.axis_index("c")
    sid = lax.axis_index("s")
    wid = cid * _NS + sid
    pltpu.sync_copy(src_hbm.at[pl.ds(wid * _CPT, _CPT)], src_v)

    def start(j, rows, gsem):
        pltpu.async_copy(tab_hbm.at[src_v.at[j]], rows, gsem)

    def wait_g(rows, gsem):
        pltpu.make_async_copy(tab_hbm.at[src_v.at[0]], rows, gsem).wait()

    def start_w(j, rows, wsem):
        base_e = (wid * _CPT + j) * _CHUNK
        pltpu.async_copy(rows, out_hbm.at[pl.ds(base_e, _CHUNK)], wsem)

    def wait_w(rows, wsem):
        pltpu.make_async_copy(rows, out_hbm.at[pl.ds(0, _CHUNK)], wsem).wait()

    start(0, rows_a, gs_a)
    start(1, rows_b, gs_b)

    def body(jj, carry):
        j0 = 2 * jj
        wait_g(rows_a, gs_a)
        start_w(j0, rows_a, ws_a)
        wait_g(rows_b, gs_b)
        start_w(j0 + 1, rows_b, ws_b)

        @pl.when(jj < _CPT // 2 - 1)
        def _():
            wait_w(rows_a, ws_a)
            start(j0 + 2, rows_a, gs_a)
            wait_w(rows_b, ws_b)
            start(j0 + 3, rows_b, gs_b)

        return carry

    lax.fori_loop(0, _CPT // 2, body, 0)
    wait_w(rows_a, ws_a)
    wait_w(rows_b, ws_b)


def _make_scatter():
    mesh = plsc.VectorSubcoreMesh(core_axis_name="c", subcore_axis_name="s")
    return functools.partial(
        pl.kernel,
        out_type=jax.ShapeDtypeStruct((_NC, _NROWS, _H), jnp.float32),
        mesh=mesh,
        scratch_types=[
            pltpu.VMEM((_CPT, _CHUNK), jnp.int32),
            pltpu.VMEM((_CHUNK, _H), jnp.float32),
            pltpu.VMEM((_CHUNK, _H), jnp.float32),
            pltpu.VMEM((_RPT, _H), jnp.float32),
            pltpu.VMEM_SHARED((_NROWS, _H), jnp.float32),
            pltpu.SemaphoreType.DMA,
            pltpu.SemaphoreType.DMA,
            pltpu.SemaphoreType.DMA,
            pltpu.SemaphoreType.DMA,
        ],
        compiler_params=pltpu.CompilerParams(use_tc_tiling_on_sc=False),
    )(_scatter_body)


def _scatter_body(msg_hbm, dst_hbm, out_hbm,
                  dst_v, msg_a, msg_b, zb_v, acc_sh, ls_a, ls_b, ss_a, ss_b):
    cid = lax.axis_index("c")
    sid = lax.axis_index("s")
    wid = cid * _NS + sid
    pltpu.sync_copy(dst_hbm.at[pl.ds(wid * _CPT, _CPT)], dst_v)

    def start_l(j, msg, lsem):
        base_e = (wid * _CPT + j) * _CHUNK
        pltpu.async_copy(msg_hbm.at[pl.ds(base_e, _CHUNK)], msg, lsem)

    def wait_l(msg, lsem):
        pltpu.make_async_copy(msg_hbm.at[pl.ds(0, _CHUNK)], msg, lsem).wait()

    def start_s(j, msg, ssem):
        pltpu.async_copy(msg, acc_sh.at[dst_v.at[j]], ssem, add=True)

    def wait_s(msg, ssem):
        pltpu.make_async_copy(msg, acc_sh.at[dst_v.at[0]], ssem).wait()

    start_l(0, msg_a, ls_a)
    start_l(1, msg_b, ls_b)

    zeros16 = jnp.zeros((_H,), jnp.float32)

    def zinit(i, carry):
        zb_v[i, :] = zeros16
        return carry

    lax.fori_loop(0, _RPT, zinit, 0)
    pltpu.sync_copy(zb_v, acc_sh.at[pl.ds(sid * _RPT, _RPT)])
    plsc.subcore_barrier()

    def body(jj, carry):
        j0 = 2 * jj
        wait_l(msg_a, ls_a)
        start_s(j0, msg_a, ss_a)
        wait_l(msg_b, ls_b)
        start_s(j0 + 1, msg_b, ss_b)

        @pl.when(jj < _CPT // 2 - 1)
        def _():
            wait_s(msg_a, ss_a)
            start_l(j0 + 2, msg_a, ls_a)
            wait_s(msg_b, ss_b)
            start_l(j0 + 3, msg_b, ls_b)

        return carry

    lax.fori_loop(0, _CPT // 2, body, 0)
    wait_s(msg_a, ss_a)
    wait_s(msg_b, ss_b)
    plsc.subcore_barrier()
    pltpu.sync_copy(acc_sh.at[pl.ds(sid * _RPT, _RPT)], zb_v)
    pltpu.sync_copy(zb_v, out_hbm.at[cid, pl.ds(sid * _RPT, _RPT)])


@functools.cache
def _sc_kernels():
    return _make_gather(_D), _make_gather(_H), _make_scatter()


def kernel(x, edge_index, edge_attr, batch,
           lin1_w, lin1_b, root1, bias1,
           lin2_w, lin2_b, root2, bias2,
           lin3_w, lin3_b, root3, bias3,
           fc1_w, fc1_b, fc2_w, fc2_b, fc3_w, fc3_b, fc4_w, fc4_b):
    gather_x, gather_h, scatter = _sc_kernels()
    pad = _EPAD - _E
    src = jnp.concatenate([edge_index[0], jnp.zeros((pad,), jnp.int32)])
    src = src.reshape(_EPAD // _CHUNK, _CHUNK)
    dst = jnp.concatenate([edge_index[1], jnp.full((pad,), _N, jnp.int32)])
    dst = dst.reshape(_EPAD // _CHUNK, _CHUNK)
    eat = jnp.concatenate(
        [edge_attr, jnp.zeros((pad, _DE), jnp.float32)], axis=0).T

    def b8(b):
        return jnp.pad(b.reshape(-1, 1), ((0, 0), (0, 7)))

    hr = x
    h = x
    for lw, lb, rt, bs, din, last in (
            (lin1_w, lin1_b, root1, bias1, _D, False),
            (lin2_w, lin2_b, root2, bias2, _H, False),
            (lin3_w, lin3_b, root3, bias3, _H, True)):
        xg = (gather_x if din == _D else gather_h)(hr, src)
        msg = _tc_einsum(din, xg, eat, lw, b8(lb))
        parts = scatter(msg, dst)
        nxt = _tc_node(parts, h, rt, bs.reshape(1, _H), rnd=not last)
        hr, h = nxt, nxt
    out8 = _tc_final(h, batch.reshape(1, _N),
                     fc1_w.T, fc1_b.reshape(1, -1),
                     fc2_w.T, fc2_b.reshape(1, -1),
                     fc3_w.T, fc3_b.reshape(1, -1),
                     jnp.pad(fc4_w.T, ((0, 0), (0, 7))),
                     jnp.pad(fc4_b, (0, 7)).reshape(1, 8))
    return out8[:, 0]


# speedup vs baseline: 1.3549x; 1.0539x over previous
import functools

import jax
import jax.numpy as jnp
from jax import lax
from jax.experimental import pallas as pl
from jax.experimental.pallas import tpu as pltpu
from jax.experimental.pallas import tpu_sc as plsc

_N = 10000
_E = 160000
_D = 128
_DE = 16
_H = 16
_G = 64

_NC = 2
_NS = 16
_NW = _NC * _NS
_CHUNK = 128
_CPT = 40
_EPAD = _NW * _CPT * _CHUNK
_NROWS = 10112
_RPT = _NROWS // _NS
_EB = 4096


def _bf16_dot(a, b):
    return jnp.dot(a.astype(jnp.bfloat16), b.astype(jnp.bfloat16),
                   preferred_element_type=jnp.float32)


def _bround(a):
    return a.astype(jnp.bfloat16).astype(jnp.float32)


def _einsum_body(din, xg_ref, eat_ref, w_ref, b_ref, msg_ref):
    wt = _bround(_bf16_dot(w_ref[...], eat_ref[...]) + b_ref[:, 0:1])
    xt = _bround(jnp.transpose(xg_ref[...]))
    acc = xt[0:1, :] * wt[0:_H, :]
    for i in range(1, din):
        acc = acc + xt[i:i + 1, :] * wt[i * _H:(i + 1) * _H, :]
    msg_ref[...] = jnp.transpose(acc)


def _tc_einsum(din, xg, eat, lin_w, lin_b8):
    return pl.pallas_call(
        functools.partial(_einsum_body, din),
        grid=(_EPAD // _EB,),
        in_specs=[
            pl.BlockSpec((_EB, din), lambda i: (i, 0)),
            pl.BlockSpec((_DE, _EB), lambda i: (0, i)),
            pl.BlockSpec((din * _H, _DE), lambda i: (0, 0)),
            pl.BlockSpec((din * _H, 8), lambda i: (0, 0)),
        ],
        out_specs=pl.BlockSpec((_EB, _H), lambda i: (i, 0)),
        out_shape=jax.ShapeDtypeStruct((_EPAD, _H), jnp.float32),
    )(xg, eat, lin_w, lin_b8)


def _node_body(rnd, p_ref, h_ref, root_ref, b_ref, out_ref):
    h = jax.nn.relu(p_ref[0, :_N] + p_ref[1, :_N]
                    + _bf16_dot(h_ref[...], root_ref[...]) + b_ref[...])
    out_ref[...] = _bround(h) if rnd else h


def _tc_node(parts, h, root, bias, rnd):
    return pl.pallas_call(
        functools.partial(_node_body, rnd),
        out_shape=jax.ShapeDtypeStruct((_N, _H), jnp.float32),
    )(parts, h, root, bias)


def _final_body(h_ref, batch_ref,
                w1_ref, b1_ref, w2_ref, b2_ref, w3_ref, b3_ref, w4_ref, b4_ref,
                out_ref):
    seg = lax.broadcasted_iota(jnp.int32, (_G, _N), 0)
    onehot = (seg == batch_ref[0][None, :]).astype(jnp.float32)
    g = jnp.dot(onehot, h_ref[...], preferred_element_type=jnp.float32,
                precision=lax.Precision.HIGHEST)
    g = jax.nn.relu(_bf16_dot(g, w1_ref[...]) + b1_ref[...])
    g = jax.nn.relu(_bf16_dot(g, w2_ref[...]) + b2_ref[...])
    g = jax.nn.relu(_bf16_dot(g, w3_ref[...]) + b3_ref[...])
    out_ref[...] = _bf16_dot(g, w4_ref[...]) + b4_ref[...]


def _tc_final(h3, batch2d, w1, b1, w2, b2, w3, b3, w4, b4):
    return pl.pallas_call(
        _final_body,
        out_shape=jax.ShapeDtypeStruct((_G, 8), jnp.float32),
    )(h3, batch2d, w1, b1, w2, b2, w3, b3, w4, b4)


def _make_gather(dw):
    mesh = plsc.VectorSubcoreMesh(core_axis_name="c", subcore_axis_name="s")
    return functools.partial(
        pl.kernel,
        out_type=jax.ShapeDtypeStruct((_EPAD, dw), jnp.float32),
        mesh=mesh,
        scratch_types=[
            pltpu.VMEM((_CPT, _CHUNK), jnp.int32),
            pltpu.VMEM((_CHUNK, dw), jnp.float32),
            pltpu.VMEM((_CHUNK, dw), jnp.float32),
            pltpu.SemaphoreType.DMA,
            pltpu.SemaphoreType.DMA,
            pltpu.SemaphoreType.DMA,
            pltpu.SemaphoreType.DMA,
        ],
        compiler_params=pltpu.CompilerParams(use_tc_tiling_on_sc=False),
    )(_gather_body)


def _gather_body(tab_hbm, src_hbm, out_hbm,
                 src_v, rows_a, rows_b, gs_a, gs_b, ws_a, ws_b):
    cid = lax.axis_index("c")
    sid = lax.axis_index("s")
    wid = cid * _NS + sid
    pltpu.sync_copy(src_hbm.at[pl.ds(wid * _CPT, _CPT)], src_v)

    def start(j, rows, gsem):
        pltpu.async_copy(tab_hbm.at[src_v.at[j]], rows, gsem)

    def wait_g(rows, gsem):
        pltpu.make_async_copy(tab_hbm.at[src_v.at[0]], rows, gsem).wait()

    def start_w(j, rows, wsem):
        base_e = (wid * _CPT + j) * _CHUNK
        pltpu.async_copy(rows, out_hbm.at[pl.ds(base_e, _CHUNK)], wsem)

    def wait_w(rows, wsem):
        pltpu.make_async_copy(rows, out_hbm.at[pl.ds(0, _CHUNK)], wsem).wait()

    start(0, rows_a, gs_a)
    start(1, rows_b, gs_b)

    def body(jj, carry):
        j0 = 2 * jj
        wait_g(rows_a, gs_a)
        start_w(j0, rows_a, ws_a)
        wait_g(rows_b, gs_b)
        start_w(j0 + 1, rows_b, ws_b)

        @pl.when(jj < _CPT // 2 - 1)
        def _():
            wait_w(rows_a, ws_a)
            start(j0 + 2, rows_a, gs_a)
            wait_w(rows_b, ws_b)
            start(j0 + 3, rows_b, gs_b)

        return carry

    lax.fori_loop(0, _CPT // 2, body, 0)
    wait_w(rows_a, ws_a)
    wait_w(rows_b, ws_b)


def _make_scatter():
    mesh = plsc.VectorSubcoreMesh(core_axis_name="c", subcore_axis_name="s")
    return functools.partial(
        pl.kernel,
        out_type=jax.ShapeDtypeStruct((_NC, _NROWS, _H), jnp.float32),
        mesh=mesh,
        scratch_types=[
            pltpu.VMEM((_CPT, _CHUNK), jnp.int32),
            pltpu.VMEM((_CHUNK, _H), jnp.float32),
            pltpu.VMEM((_CHUNK, _H), jnp.float32),
            pltpu.VMEM((_RPT, _H), jnp.float32),
            pltpu.VMEM_SHARED((_NROWS, _H), jnp.float32),
            pltpu.SemaphoreType.DMA,
            pltpu.SemaphoreType.DMA,
            pltpu.SemaphoreType.DMA,
            pltpu.SemaphoreType.DMA,
        ],
        compiler_params=pltpu.CompilerParams(use_tc_tiling_on_sc=False),
    )(_scatter_body)


def _scatter_body(msg_hbm, dst_hbm, out_hbm,
                  dst_v, msg_a, msg_b, zb_v, acc_sh, ls_a, ls_b, ss_a, ss_b):
    cid = lax.axis_index("c")
    sid = lax.axis_index("s")
    wid = cid * _NS + sid
    pltpu.sync_copy(dst_hbm.at[pl.ds(wid * _CPT, _CPT)], dst_v)

    def start_l(j, msg, lsem):
        base_e = (wid * _CPT + j) * _CHUNK
        pltpu.async_copy(msg_hbm.at[pl.ds(base_e, _CHUNK)], msg, lsem)

    def wait_l(msg, lsem):
        pltpu.make_async_copy(msg_hbm.at[pl.ds(0, _CHUNK)], msg, lsem).wait()

    def start_s(j, msg, ssem):
        pltpu.async_copy(msg, acc_sh.at[dst_v.at[j]], ssem, add=True)

    def wait_s(msg, ssem):
        pltpu.make_async_copy(msg, acc_sh.at[dst_v.at[0]], ssem).wait()

    start_l(0, msg_a, ls_a)
    start_l(1, msg_b, ls_b)

    zeros16 = jnp.zeros((_H,), jnp.float32)

    def zinit(i, carry):
        zb_v[i, :] = zeros16
        return carry

    lax.fori_loop(0, _RPT, zinit, 0)
    pltpu.sync_copy(zb_v, acc_sh.at[pl.ds(sid * _RPT, _RPT)])
    plsc.subcore_barrier()

    def body(jj, carry):
        j0 = 2 * jj
        wait_l(msg_a, ls_a)
        start_s(j0, msg_a, ss_a)
        wait_l(msg_b, ls_b)
        start_s(j0 + 1, msg_b, ss_b)

        @pl.when(jj < _CPT // 2 - 1)
        def _():
            wait_s(msg_a, ss_a)
            start_l(j0 + 2, msg_a, ls_a)
            wait_s(msg_b, ss_b)
            start_l(j0 + 3, msg_b, ls_b)

        return carry

    lax.fori_loop(0, _CPT // 2, body, 0)
    wait_s(msg_a, ss_a)
    wait_s(msg_b, ss_b)
    plsc.subcore_barrier()
    pltpu.sync_copy(acc_sh.at[pl.ds(sid * _RPT, _RPT)], zb_v)
    pltpu.sync_copy(zb_v, out_hbm.at[cid, pl.ds(sid * _RPT, _RPT)])


@functools.cache
def _sc_kernels():
    return _make_gather(_D), _make_gather(_H), _make_scatter()


def kernel(x, edge_index, edge_attr, batch,
           lin1_w, lin1_b, root1, bias1,
           lin2_w, lin2_b, root2, bias2,
           lin3_w, lin3_b, root3, bias3,
           fc1_w, fc1_b, fc2_w, fc2_b, fc3_w, fc3_b, fc4_w, fc4_b):
    gather_x, gather_h, scatter = _sc_kernels()
    pad = _EPAD - _E
    src = jnp.concatenate([edge_index[0], jnp.zeros((pad,), jnp.int32)])
    src = src.reshape(_EPAD // _CHUNK, _CHUNK)
    dst = jnp.concatenate([edge_index[1], jnp.full((pad,), _N, jnp.int32)])
    dst = dst.reshape(_EPAD // _CHUNK, _CHUNK)
    eat = jnp.concatenate(
        [edge_attr, jnp.zeros((pad, _DE), jnp.float32)], axis=0).T

    def b8(b):
        return jnp.pad(b.reshape(-1, 1), ((0, 0), (0, 7)))

    hr = x
    h = x
    for lw, lb, rt, bs, din, last in (
            (lin1_w, lin1_b, root1, bias1, _D, False),
            (lin2_w, lin2_b, root2, bias2, _H, False),
            (lin3_w, lin3_b, root3, bias3, _H, True)):
        xg = (gather_x if din == _D else gather_h)(hr, src)
        msg = _tc_einsum(din, xg, eat, lw, b8(lb))
        parts = scatter(msg, dst)
        nxt = _tc_node(parts, h, rt, bs.reshape(1, _H), rnd=not last)
        hr, h = nxt, nxt
    out8 = _tc_final(h, batch.reshape(1, _N),
                     fc1_w.T, fc1_b.reshape(1, -1),
                     fc2_w.T, fc2_b.reshape(1, -1),
                     fc3_w.T, fc3_b.reshape(1, -1),
                     jnp.pad(fc4_w.T, ((0, 0), (0, 7))),
                     jnp.pad(fc4_b, (0, 7)).reshape(1, 8))
    return out8[:, 0]


# speedup vs baseline: 1.3703x; 1.0113x over previous
import functools

import jax
import jax.numpy as jnp
from jax import lax
from jax.experimental import pallas as pl
from jax.experimental.pallas import tpu as pltpu
from jax.experimental.pallas import tpu_sc as plsc

_N = 10000
_E = 160000
_D = 128
_DE = 16
_H = 16
_G = 64

_NC = 2
_NS = 16
_NW = _NC * _NS
_CHUNK = 128
_CPT = 40
_EPAD = _NW * _CPT * _CHUNK
_NROWS = 10112
_RPT = _NROWS // _NS
_EB = 4096


def _bf16_dot(a, b):
    return jnp.dot(a.astype(jnp.bfloat16), b.astype(jnp.bfloat16),
                   preferred_element_type=jnp.float32)


def _bround(a):
    return a.astype(jnp.bfloat16).astype(jnp.float32)


def _einsum_body(din, xg_ref, eat_ref, w_ref, b_ref, msg_ref):
    wt = _bround(_bf16_dot(w_ref[...], eat_ref[...]) + b_ref[:, 0:1])
    xt = _bround(jnp.transpose(xg_ref[...]))
    acc = xt[0:1, :] * wt[0:_H, :]
    for i in range(1, din):
        acc = acc + xt[i:i + 1, :] * wt[i * _H:(i + 1) * _H, :]
    msg_ref[...] = jnp.transpose(acc)


def _tc_einsum(din, xg, eat, lin_w, lin_b8):
    return pl.pallas_call(
        functools.partial(_einsum_body, din),
        grid=(_EPAD // _EB,),
        in_specs=[
            pl.BlockSpec((_EB, din), lambda i: (i, 0)),
            pl.BlockSpec((_DE, _EB), lambda i: (0, i)),
            pl.BlockSpec((din * _H, _DE), lambda i: (0, 0)),
            pl.BlockSpec((din * _H, 8), lambda i: (0, 0)),
        ],
        out_specs=pl.BlockSpec((_EB, _H), lambda i: (i, 0)),
        out_shape=jax.ShapeDtypeStruct((_EPAD, _H), jnp.float32),
    )(xg, eat, lin_w, lin_b8)


def _node_body(rnd, p_ref, h_ref, root_ref, b_ref, out_ref):
    h = jax.nn.relu(p_ref[0, :_N] + p_ref[1, :_N]
                    + _bf16_dot(h_ref[...], root_ref[...]) + b_ref[...])
    out_ref[...] = _bround(h) if rnd else h


def _tc_node(parts, h, root, bias, rnd):
    return pl.pallas_call(
        functools.partial(_node_body, rnd),
        out_shape=jax.ShapeDtypeStruct((_N, _H), jnp.float32),
    )(parts, h, root, bias)


def _final_body(h_ref, batch_ref,
                w1_ref, b1_ref, w2_ref, b2_ref, w3_ref, b3_ref, w4_ref, b4_ref,
                out_ref):
    seg = lax.broadcasted_iota(jnp.int32, (_G, _N), 0)
    onehot = (seg == batch_ref[0][None, :]).astype(jnp.float32)
    g = jnp.dot(onehot, h_ref[...], preferred_element_type=jnp.float32,
                precision=lax.Precision.HIGHEST)
    g = jax.nn.relu(_bf16_dot(g, w1_ref[...]) + b1_ref[...])
    g = jax.nn.relu(_bf16_dot(g, w2_ref[...]) + b2_ref[...])
    g = jax.nn.relu(_bf16_dot(g, w3_ref[...]) + b3_ref[...])
    out_ref[...] = _bf16_dot(g, w4_ref[...]) + b4_ref[...]


def _tc_final(h3, batch2d, w1, b1, w2, b2, w3, b3, w4, b4):
    return pl.pallas_call(
        _final_body,
        out_shape=jax.ShapeDtypeStruct((_G, 8), jnp.float32),
    )(h3, batch2d, w1, b1, w2, b2, w3, b3, w4, b4)


def _make_gather(dw):
    mesh = plsc.VectorSubcoreMesh(core_axis_name="c", subcore_axis_name="s")
    return functools.partial(
        pl.kernel,
        out_type=jax.ShapeDtypeStruct((_EPAD, dw), jnp.float32),
        mesh=mesh,
        scratch_types=[
            pltpu.VMEM((_CPT, _CHUNK), jnp.int32),
            pltpu.VMEM((_CHUNK, dw), jnp.float32),
            pltpu.VMEM((_CHUNK, dw), jnp.float32),
            pltpu.VMEM((_CHUNK, dw), jnp.float32),
            pltpu.VMEM((_CHUNK, dw), jnp.float32),
            pltpu.SemaphoreType.DMA,
            pltpu.SemaphoreType.DMA,
            pltpu.SemaphoreType.DMA,
            pltpu.SemaphoreType.DMA,
            pltpu.SemaphoreType.DMA,
            pltpu.SemaphoreType.DMA,
            pltpu.SemaphoreType.DMA,
            pltpu.SemaphoreType.DMA,
        ],
        compiler_params=pltpu.CompilerParams(use_tc_tiling_on_sc=False),
    )(_gather_body)


def _gather_body(tab_hbm, src_hbm, out_hbm, src_v,
                 r0, r1, r2, r3, g0, g1, g2, g3, w0, w1, w2, w3):
    cid = lax.axis_index("c")
    sid = lax.axis_index("s")
    wid = cid * _NS + sid
    pltpu.sync_copy(src_hbm.at[pl.ds(wid * _CPT, _CPT)], src_v)
    bufs = (r0, r1, r2, r3)
    gsems = (g0, g1, g2, g3)
    wsems = (w0, w1, w2, w3)

    def start(j, b):
        pltpu.async_copy(tab_hbm.at[src_v.at[j]], bufs[b], gsems[b])

    def wait_g(b):
        pltpu.make_async_copy(tab_hbm.at[src_v.at[0]], bufs[b],
                              gsems[b]).wait()

    def start_w(j, b):
        base_e = (wid * _CPT + j) * _CHUNK
        pltpu.async_copy(bufs[b], out_hbm.at[pl.ds(base_e, _CHUNK)], wsems[b])

    def wait_w(b):
        pltpu.make_async_copy(bufs[b], out_hbm.at[pl.ds(0, _CHUNK)],
                              wsems[b]).wait()

    for b in range(4):
        start(b, b)

    def body(jj, carry):
        j0 = 4 * jj
        for b in range(4):
            wait_g(b)
            start_w(j0 + b, b)

        @pl.when(jj < _CPT // 4 - 1)
        def _():
            for b in range(4):
                wait_w(b)
                start(j0 + 4 + b, b)

        return carry

    lax.fori_loop(0, _CPT // 4, body, 0)
    for b in range(4):
        wait_w(b)


def _make_scatter():
    mesh = plsc.VectorSubcoreMesh(core_axis_name="c", subcore_axis_name="s")
    return functools.partial(
        pl.kernel,
        out_type=jax.ShapeDtypeStruct((_NC, _NROWS, _H), jnp.float32),
        mesh=mesh,
        scratch_types=[
            pltpu.VMEM((_CPT, _CHUNK), jnp.int32),
            pltpu.VMEM((_CHUNK, _H), jnp.float32),
            pltpu.VMEM((_CHUNK, _H), jnp.float32),
            pltpu.VMEM((_RPT, _H), jnp.float32),
            pltpu.VMEM_SHARED((_NROWS, _H), jnp.float32),
            pltpu.SemaphoreType.DMA,
            pltpu.SemaphoreType.DMA,
            pltpu.SemaphoreType.DMA,
            pltpu.SemaphoreType.DMA,
        ],
        compiler_params=pltpu.CompilerParams(use_tc_tiling_on_sc=False),
    )(_scatter_body)


def _scatter_body(msg_hbm, dst_hbm, out_hbm,
                  dst_v, msg_a, msg_b, zb_v, acc_sh, ls_a, ls_b, ss_a, ss_b):
    cid = lax.axis_index("c")
    sid = lax.axis_index("s")
    wid = cid * _NS + sid
    pltpu.sync_copy(dst_hbm.at[pl.ds(wid * _CPT, _CPT)], dst_v)

    def start_l(j, msg, lsem):
        base_e = (wid * _CPT + j) * _CHUNK
        pltpu.async_copy(msg_hbm.at[pl.ds(base_e, _CHUNK)], msg, lsem)

    def wait_l(msg, lsem):
        pltpu.make_async_copy(msg_hbm.at[pl.ds(0, _CHUNK)], msg, lsem).wait()

    def start_s(j, msg, ssem):
        pltpu.async_copy(msg, acc_sh.at[dst_v.at[j]], ssem, add=True)

    def wait_s(msg, ssem):
        pltpu.make_async_copy(msg, acc_sh.at[dst_v.at[0]], ssem).wait()

    start_l(0, msg_a, ls_a)
    start_l(1, msg_b, ls_b)

    zeros16 = jnp.zeros((_H,), jnp.float32)

    def zinit(i, carry):
        zb_v[i, :] = zeros16
        return carry

    lax.fori_loop(0, _RPT, zinit, 0)
    pltpu.sync_copy(zb_v, acc_sh.at[pl.ds(sid * _RPT, _RPT)])
    plsc.subcore_barrier()

    def body(jj, carry):
        j0 = 2 * jj
        wait_l(msg_a, ls_a)
        start_s(j0, msg_a, ss_a)
        wait_l(msg_b, ls_b)
        start_s(j0 + 1, msg_b, ss_b)

        @pl.when(jj < _CPT // 2 - 1)
        def _():
            wait_s(msg_a, ss_a)
            start_l(j0 + 2, msg_a, ls_a)
            wait_s(msg_b, ss_b)
            start_l(j0 + 3, msg_b, ls_b)

        return carry

    lax.fori_loop(0, _CPT // 2, body, 0)
    wait_s(msg_a, ss_a)
    wait_s(msg_b, ss_b)
    plsc.subcore_barrier()
    pltpu.sync_copy(acc_sh.at[pl.ds(sid * _RPT, _RPT)], zb_v)
    pltpu.sync_copy(zb_v, out_hbm.at[cid, pl.ds(sid * _RPT, _RPT)])


@functools.cache
def _sc_kernels():
    return _make_gather(_D), _make_gather(_H), _make_scatter()


def kernel(x, edge_index, edge_attr, batch,
           lin1_w, lin1_b, root1, bias1,
           lin2_w, lin2_b, root2, bias2,
           lin3_w, lin3_b, root3, bias3,
           fc1_w, fc1_b, fc2_w, fc2_b, fc3_w, fc3_b, fc4_w, fc4_b):
    gather_x, gather_h, scatter = _sc_kernels()
    pad = _EPAD - _E
    src = jnp.concatenate([edge_index[0], jnp.zeros((pad,), jnp.int32)])
    src = src.reshape(_EPAD // _CHUNK, _CHUNK)
    dst = jnp.concatenate([edge_index[1], jnp.full((pad,), _N, jnp.int32)])
    dst = dst.reshape(_EPAD // _CHUNK, _CHUNK)
    eat = jnp.concatenate(
        [edge_attr, jnp.zeros((pad, _DE), jnp.float32)], axis=0).T

    def b8(b):
        return jnp.pad(b.reshape(-1, 1), ((0, 0), (0, 7)))

    hr = x
    h = x
    for lw, lb, rt, bs, din, last in (
            (lin1_w, lin1_b, root1, bias1, _D, False),
            (lin2_w, lin2_b, root2, bias2, _H, False),
            (lin3_w, lin3_b, root3, bias3, _H, True)):
        xg = (gather_x if din == _D else gather_h)(hr, src)
        msg = _tc_einsum(din, xg, eat, lw, b8(lb))
        parts = scatter(msg, dst)
        nxt = _tc_node(parts, h, rt, bs.reshape(1, _H), rnd=not last)
        hr, h = nxt, nxt
    out8 = _tc_final(h, batch.reshape(1, _N),
                     fc1_w.T, fc1_b.reshape(1, -1),
                     fc2_w.T, fc2_b.reshape(1, -1),
                     fc3_w.T, fc3_b.reshape(1, -1),
                     jnp.pad(fc4_w.T, ((0, 0), (0, 7))),
                     jnp.pad(fc4_b, (0, 7)).reshape(1, 8))
    return out8[:, 0]
